# Initial kernel scaffold; baseline (speedup 1.0000x reference)
import jax
import jax.numpy as jnp
from jax.experimental import pallas as pl


def kernel(x, edge_index, base_w1, spline_w1, scaler1, base_w2, spline_w2, scaler2, base_wo, spline_wo, scaler_o):
    raise NotImplementedError("write your pallas kernel here")


# speedup vs baseline: 2.2623x; 2.2623x over previous
import jax
import jax.numpy as jnp
from jax.experimental import pallas as pl

_GRID_SIZE = 4
_ORDER = 3
_H = 0.5
_KNOTS = [_H * i - 2.5 for i in range(_GRID_SIZE + 2 * _ORDER + 1)]


def _spline_bases(y):
    nb = len(_KNOTS) - 1
    b = [((y >= _KNOTS[i]) & (y < _KNOTS[i + 1])).astype(jnp.float32)
         for i in range(nb)]
    for j in range(1, _ORDER + 1):
        inv = 1.0 / (j * _H)
        b = [(y - _KNOTS[i]) * inv * b[i]
             + (_KNOTS[i + j + 1] - y) * inv * b[i + 1]
             for i in range(nb - j)]
    return b


def _kan(y, bw_ref, sw_ref):
    base = jnp.dot(jax.nn.silu(y).astype(jnp.bfloat16), bw_ref[...],
                   preferred_element_type=jnp.float32)
    bs = jnp.concatenate(_spline_bases(y), axis=1).astype(jnp.bfloat16)
    spline = jnp.dot(bs, sw_ref[...], preferred_element_type=jnp.float32)
    return base + spline


def _pass12_kernel(a_ref, f_ref, bw_ref, sw_ref, y_ref, h16_ref):
    a16 = a_ref[...].astype(jnp.bfloat16)
    y = jnp.dot(a16, f_ref[...], preferred_element_type=jnp.float32)
    y_ref[...] = y
    h = jnp.maximum(_kan(y, bw_ref, sw_ref), 0.0)
    h16_ref[...] = h.astype(jnp.bfloat16)


def _pass3_kernel(a_ref, f_ref, y1_ref, y2_ref, bw_ref, sw_ref, o_ref):
    a16 = a_ref[...].astype(jnp.bfloat16)
    y3 = jnp.dot(a16, f_ref[...], preferred_element_type=jnp.float32)
    yc = jnp.concatenate([y1_ref[...], y2_ref[...], y3], axis=1)
    o_ref[...] = jnp.maximum(_kan(yc, bw_ref, sw_ref), 0.0)


def _prep_spline_w(spline_w, scaler):
    sw = spline_w * scaler[:, :, None]
    w = sw.transpose(2, 1, 0).reshape(-1, sw.shape[0])
    return w.astype(jnp.bfloat16)


def _full(shape):
    return pl.BlockSpec(shape, lambda i: (0, 0))


def kernel(x, edge_index, base_w1, spline_w1, scaler1, base_w2, spline_w2,
           scaler2, base_wo, spline_wo, scaler_o):
    n, f = x.shape
    h_dim = base_w1.shape[0]
    c_dim = base_wo.shape[0]
    bm = 200
    assert n % bm == 0
    grid = (n // bm,)

    x16 = x.astype(jnp.bfloat16)
    bw1 = base_w1.T.astype(jnp.bfloat16)
    bw2 = base_w2.T.astype(jnp.bfloat16)
    bwo = base_wo.T.astype(jnp.bfloat16)
    sw1 = _prep_spline_w(spline_w1, scaler1)
    sw2 = _prep_spline_w(spline_w2, scaler2)
    swo = _prep_spline_w(spline_wo, scaler_o)

    row_blk = pl.BlockSpec((bm, n), lambda i: (i, 0))

    def layer12(feat16, bw, sw, fin):
        return pl.pallas_call(
            _pass12_kernel,
            grid=grid,
            in_specs=[row_blk, _full((n, fin)), _full(bw.shape), _full(sw.shape)],
            out_specs=[pl.BlockSpec((bm, h_dim), lambda i: (i, 0)),
                       pl.BlockSpec((bm, h_dim), lambda i: (i, 0))],
            out_shape=[jax.ShapeDtypeStruct((n, h_dim), jnp.float32),
                       jax.ShapeDtypeStruct((n, h_dim), jnp.bfloat16)],
        )(edge_index, feat16, bw, sw)

    y1, h16 = layer12(x16, bw1, sw1, f)
    y2, h2_16 = layer12(h16, bw2, sw2, h_dim)

    out = pl.pallas_call(
        _pass3_kernel,
        grid=grid,
        in_specs=[row_blk, _full((n, h_dim)),
                  pl.BlockSpec((bm, h_dim), lambda i: (i, 0)),
                  pl.BlockSpec((bm, h_dim), lambda i: (i, 0)),
                  _full(bwo.shape), _full(swo.shape)],
        out_specs=pl.BlockSpec((bm, c_dim), lambda i: (i, 0)),
        out_shape=jax.ShapeDtypeStruct((n, c_dim), jnp.float32),
    )(edge_index, h2_16, y1, y2, bwo, swo)
    return out
